# Initial kernel scaffold; baseline (speedup 1.0000x reference)
import jax
import jax.numpy as jnp
from jax.experimental import pallas as pl


def kernel(local, chain, batch, mask, W1, b1, W2, b2):
    raise NotImplementedError("write your pallas kernel here")


# speedup vs baseline: 9.0904x; 9.0904x over previous
import functools

import jax
import jax.numpy as jnp
from jax import lax
from jax.experimental import pallas as pl

_NUM_BATCH = 16
_NUM_CHAIN = 128
_NSEG = _NUM_BATCH + _NUM_CHAIN


def _pick_block(n):
    for r in (3200, 6400, 1600, 800, 400, 320, 160, 80, 40, 16, 8):
        if n % r == 0:
            return r
    return n


def _reduce_body(local_ref, batch_ref, chain_ref, mask_ref, s_ref, cnt_ref):
    @pl.when(pl.program_id(0) == 0)
    def _init():
        s_ref[...] = jnp.zeros_like(s_ref)
        cnt_ref[...] = jnp.zeros_like(cnt_ref)

    x = local_ref[...]
    b = batch_ref[0, 0, :]
    c = chain_ref[0, 0, :]
    m = mask_ref[0, 0, :]
    r = x.shape[0]
    seg_iota = lax.broadcasted_iota(jnp.int32, (r, _NSEG), 1)
    onehot = ((b[:, None] == seg_iota).astype(jnp.float32)
              + (c[:, None] == (seg_iota - _NUM_BATCH)).astype(jnp.float32)
              ) * m[:, None]
    s_ref[...] += lax.dot_general(
        onehot, x, (((0,), (0,)), ((), ())),
        preferred_element_type=jnp.float32,
        precision=lax.Precision.HIGHEST)
    ones = jnp.ones((r, cnt_ref.shape[1]), dtype=jnp.float32)
    cnt_ref[...] += lax.dot_general(
        onehot, ones, (((0,), (0,)), ((), ())),
        preferred_element_type=jnp.float32,
        precision=lax.Precision.HIGHEST)


def _middle_body(s_ref, cnt_ref, w1_ref, b1_ref, w2_ref, a_ref, bb_ref):
    mean = s_ref[...] / jnp.maximum(cnt_ref[...], 1e-6)
    u = lax.dot_general(
        mean, w1_ref[...], (((1,), (0,)), ((), ())),
        preferred_element_type=jnp.float32,
        precision=lax.Precision.HIGHEST) + b1_ref[0, :]
    g = jax.nn.gelu(u)
    a_ref[...] = lax.dot_general(
        g[:_NUM_BATCH], w2_ref[: u.shape[1]],
        (((1,), (0,)), ((), ())),
        preferred_element_type=jnp.float32,
        precision=lax.Precision.HIGHEST)
    bb_ref[...] = lax.dot_general(
        g[_NUM_BATCH:], w2_ref[u.shape[1]:],
        (((1,), (0,)), ((), ())),
        preferred_element_type=jnp.float32,
        precision=lax.Precision.HIGHEST)


def _broadcast_body(batch_ref, chain_ref, a_ref, bb_ref, b2_ref, out_ref):
    b = batch_ref[0, 0, :]
    c = chain_ref[0, 0, :]
    r = b.shape[0]
    iota_b = lax.broadcasted_iota(jnp.int32, (r, _NUM_BATCH), 1)
    iota_c = lax.broadcasted_iota(jnp.int32, (r, _NUM_CHAIN), 1)
    ohb = (b[:, None] == iota_b).astype(jnp.float32)
    ohc = (c[:, None] == iota_c).astype(jnp.float32)
    out = lax.dot_general(
        ohb, a_ref[...], (((1,), (0,)), ((), ())),
        preferred_element_type=jnp.float32,
        precision=lax.Precision.HIGHEST)
    out += lax.dot_general(
        ohc, bb_ref[...], (((1,), (0,)), ((), ())),
        preferred_element_type=jnp.float32,
        precision=lax.Precision.HIGHEST)
    out_ref[...] = out + b2_ref[0, :]


@jax.jit
def kernel(local, chain, batch, mask, W1, b1, W2, b2):
    n, d = local.shape
    r = _pick_block(n)
    gb = n // r
    batch3 = batch.astype(jnp.int32).reshape(gb, 1, r)
    chain3 = chain.astype(jnp.int32).reshape(gb, 1, r)
    mask3 = mask.astype(jnp.float32).reshape(gb, 1, r)

    s, cnt = pl.pallas_call(
        _reduce_body,
        grid=(gb,),
        in_specs=[
            pl.BlockSpec((r, d), lambda i: (i, 0)),
            pl.BlockSpec((1, 1, r), lambda i: (i, 0, 0)),
            pl.BlockSpec((1, 1, r), lambda i: (i, 0, 0)),
            pl.BlockSpec((1, 1, r), lambda i: (i, 0, 0)),
        ],
        out_specs=[
            pl.BlockSpec((_NSEG, d), lambda i: (0, 0)),
            pl.BlockSpec((_NSEG, d), lambda i: (0, 0)),
        ],
        out_shape=[
            jax.ShapeDtypeStruct((_NSEG, d), jnp.float32),
            jax.ShapeDtypeStruct((_NSEG, d), jnp.float32),
        ],
    )(local, batch3, chain3, mask3)

    a, bb = pl.pallas_call(
        _middle_body,
        out_shape=[
            jax.ShapeDtypeStruct((_NUM_BATCH, d), jnp.float32),
            jax.ShapeDtypeStruct((_NUM_CHAIN, d), jnp.float32),
        ],
    )(s, cnt, W1, b1.reshape(1, -1), W2)

    out = pl.pallas_call(
        _broadcast_body,
        grid=(gb,),
        in_specs=[
            pl.BlockSpec((1, 1, r), lambda i: (i, 0, 0)),
            pl.BlockSpec((1, 1, r), lambda i: (i, 0, 0)),
            pl.BlockSpec((_NUM_BATCH, d), lambda i: (0, 0)),
            pl.BlockSpec((_NUM_CHAIN, d), lambda i: (0, 0)),
            pl.BlockSpec((1, d), lambda i: (0, 0)),
        ],
        out_specs=pl.BlockSpec((r, d), lambda i: (i, 0)),
        out_shape=jax.ShapeDtypeStruct((n, d), jnp.float32),
    )(batch3, chain3, a, bb, b2.reshape(1, -1))
    return out


# speedup vs baseline: 18.9942x; 2.0895x over previous
import jax
import jax.numpy as jnp
from jax import lax
from jax.experimental import pallas as pl

_NUM_BATCH = 16
_NUM_CHAIN = 128
_NSEG = _NUM_BATCH + _NUM_CHAIN


def _pick_block(n):
    for r in (3200, 6400, 1600, 800, 400, 320, 160, 80, 40, 16, 8):
        if n % r == 0:
            return r
    return n


def _onehot_bf16(b, c, r):
    seg_iota = lax.broadcasted_iota(jnp.int32, (r, _NSEG), 1)
    return ((b[:, None] == seg_iota).astype(jnp.bfloat16)
            + (c[:, None] == (seg_iota - _NUM_BATCH)).astype(jnp.bfloat16))


def _reduce_body(local_ref, batch_ref, chain_ref, mask_ref, sc_ref):
    @pl.when(pl.program_id(0) == 0)
    def _init():
        sc_ref[...] = jnp.zeros_like(sc_ref)

    x = local_ref[...]
    b = batch_ref[0, 0, :]
    c = chain_ref[0, 0, :]
    m = mask_ref[0, 0, :]
    r, d = x.shape
    onehot = _onehot_bf16(b, c, r) * m[:, None].astype(jnp.bfloat16)
    aug = jnp.concatenate(
        [x.astype(jnp.bfloat16), jnp.ones((r, d), jnp.bfloat16)], axis=1)
    sc_ref[...] += lax.dot_general(
        onehot, aug, (((0,), (0,)), ((), ())),
        preferred_element_type=jnp.float32)


def _middle_body(sc_ref, w1_ref, b1_ref, w2_ref, b2_ref, t_ref):
    d = t_ref.shape[1]
    s = sc_ref[:, :d]
    cnt = sc_ref[:, d:]
    mean = s / jnp.maximum(cnt, 1e-6)
    u = lax.dot_general(
        mean, w1_ref[...], (((1,), (0,)), ((), ())),
        preferred_element_type=jnp.float32,
        precision=lax.Precision.HIGHEST) + b1_ref[0, :]
    g = jax.nn.gelu(u)
    a = lax.dot_general(
        g[:_NUM_BATCH], w2_ref[: u.shape[1]],
        (((1,), (0,)), ((), ())),
        preferred_element_type=jnp.float32,
        precision=lax.Precision.HIGHEST)
    bb = lax.dot_general(
        g[_NUM_BATCH:], w2_ref[u.shape[1]:],
        (((1,), (0,)), ((), ())),
        preferred_element_type=jnp.float32,
        precision=lax.Precision.HIGHEST)
    t_ref[...] = jnp.concatenate([a + b2_ref[0, :], bb], axis=0)


def _broadcast_body(batch_ref, chain_ref, t_ref, out_ref):
    b = batch_ref[0, 0, :]
    c = chain_ref[0, 0, :]
    r = b.shape[0]
    onehot = _onehot_bf16(b, c, r)
    out_ref[...] = lax.dot_general(
        onehot, t_ref[...].astype(jnp.bfloat16), (((1,), (0,)), ((), ())),
        preferred_element_type=jnp.float32)


@jax.jit
def kernel(local, chain, batch, mask, W1, b1, W2, b2):
    n, d = local.shape
    r = _pick_block(n)
    gb = n // r
    batch3 = batch.astype(jnp.int32).reshape(gb, 1, r)
    chain3 = chain.astype(jnp.int32).reshape(gb, 1, r)
    mask3 = mask.astype(jnp.float32).reshape(gb, 1, r)

    sc = pl.pallas_call(
        _reduce_body,
        grid=(gb,),
        in_specs=[
            pl.BlockSpec((r, d), lambda i: (i, 0)),
            pl.BlockSpec((1, 1, r), lambda i: (i, 0, 0)),
            pl.BlockSpec((1, 1, r), lambda i: (i, 0, 0)),
            pl.BlockSpec((1, 1, r), lambda i: (i, 0, 0)),
        ],
        out_specs=pl.BlockSpec((_NSEG, 2 * d), lambda i: (0, 0)),
        out_shape=jax.ShapeDtypeStruct((_NSEG, 2 * d), jnp.float32),
    )(local, batch3, chain3, mask3)

    t = pl.pallas_call(
        _middle_body,
        out_shape=jax.ShapeDtypeStruct((_NSEG, d), jnp.float32),
    )(sc, W1, b1.reshape(1, -1), W2, b2.reshape(1, -1))

    out = pl.pallas_call(
        _broadcast_body,
        grid=(gb,),
        in_specs=[
            pl.BlockSpec((1, 1, r), lambda i: (i, 0, 0)),
            pl.BlockSpec((1, 1, r), lambda i: (i, 0, 0)),
            pl.BlockSpec((_NSEG, d), lambda i: (0, 0)),
        ],
        out_specs=pl.BlockSpec((r, d), lambda i: (i, 0)),
        out_shape=jax.ShapeDtypeStruct((n, d), jnp.float32),
    )(batch3, chain3, t)
    return out
